# baseline (speedup 1.0000x reference)
import functools

import jax
import jax.numpy as jnp
from jax import lax
from jax.experimental import pallas as pl
from jax.experimental.pallas import tpu as pltpu
from jax.experimental.pallas import tpu_sc as plsc

E = 8
K = 2
T = 4096
DIN = 1024
DOUT = 1024
TILE_M = 256
PAD_TOT = T * K + E * TILE_M
NT = PAD_TOT // TILE_M

NC, NS = 2, 16
NW = NC * NS
P_PER_W = K * T // NW

ROUTER_TILE = 512


def _sc_mesh():
    return plsc.VectorSubcoreMesh(
        core_axis_name="c", subcore_axis_name="s", num_cores=NC, num_subcores=NS)


def _router_body(x_ref, wr_ref, br_ref,
                 i0_ref, i1_ref, w0_ref, w1_ref, cnt_ref, rank_ref):
    logits = jnp.dot(x_ref[...], wr_ref[...], preferred_element_type=jnp.float32)
    logits = logits + br_ref[...]
    ids = lax.broadcasted_iota(jnp.int32, logits.shape, 1)
    m0 = jnp.max(logits, axis=1, keepdims=True)
    i0 = jnp.min(jnp.where(logits == m0, ids, E), axis=1)
    masked = jnp.where(ids == i0[:, None], -jnp.inf, logits)
    m1 = jnp.max(masked, axis=1, keepdims=True)
    i1 = jnp.min(jnp.where(masked == m1, ids, E), axis=1)
    d = jnp.exp(m1[:, 0] - m0[:, 0])
    i0_ref[...] = i0
    i1_ref[...] = i1
    w0_ref[...] = 1.0 / (1.0 + d)
    w1_ref[...] = d / (1.0 + d)
    eids = lax.broadcasted_iota(jnp.int32, (P_PER_W, E), 1)
    tri = (lax.broadcasted_iota(jnp.int32, (P_PER_W, P_PER_W), 0)
           > lax.broadcasted_iota(jnp.int32, (P_PER_W, P_PER_W), 1)
           ).astype(jnp.float32)
    for h in range(2):
        sl = slice(h * P_PER_W, (h + 1) * P_PER_W)
        for k, ik in ((0, i0), (1, i1)):
            onehot = (ik[sl][:, None] == eids).astype(jnp.float32)
            cnt_ref[0, 2 * k + h] = jnp.sum(onehot, axis=0).astype(jnp.int32)
            before = jnp.dot(tri, onehot, preferred_element_type=jnp.float32)
            rank_ref[0, 2 * k + h] = jnp.sum(
                before * onehot, axis=1).astype(jnp.int32)


def _router(flat_x, Wr, br):
    nblk = T // ROUTER_TILE
    out_shapes = (
        jax.ShapeDtypeStruct((T,), jnp.int32),
        jax.ShapeDtypeStruct((T,), jnp.int32),
        jax.ShapeDtypeStruct((T,), jnp.float32),
        jax.ShapeDtypeStruct((T,), jnp.float32),
        jax.ShapeDtypeStruct((nblk, 4, E), jnp.int32),
        jax.ShapeDtypeStruct((nblk, 4, P_PER_W), jnp.int32),
    )
    vec_spec = pl.BlockSpec((ROUTER_TILE,), lambda i: (i,))
    return pl.pallas_call(
        _router_body,
        grid=(nblk,),
        in_specs=[
            pl.BlockSpec((ROUTER_TILE, DIN), lambda i: (i, 0)),
            pl.BlockSpec((DIN, E), lambda i: (0, 0)),
            pl.BlockSpec((1, E), lambda i: (0, 0)),
        ],
        out_specs=(vec_spec, vec_spec, vec_spec, vec_spec,
                   pl.BlockSpec((1, 4, E), lambda i: (i, 0, 0)),
                   pl.BlockSpec((1, 4, P_PER_W), lambda i: (i, 0, 0))),
        out_shape=out_shapes,
    )(flat_x, Wr, br.reshape(1, E))


DCH = 16
NCH_D = P_PER_W // DCH
DRING = 6


def _sc_dispatch(w_flat, pos2d, flat_x):
    @functools.partial(
        pl.kernel,
        out_type=(
            jax.ShapeDtypeStruct((PAD_TOT, DIN), jnp.float32),
            jax.ShapeDtypeStruct((PAD_TOT,), jnp.float32),
        ),
        mesh=_sc_mesh(),
        scratch_types=(
            [pltpu.VMEM((P_PER_W,), jnp.float32),
             pltpu.VMEM((NCH_D, DCH), jnp.int32),
             pltpu.VMEM((NCH_D, DCH), jnp.float32)]
            + [pltpu.VMEM((DCH, DIN), jnp.float32)] * DRING
            + [pltpu.SemaphoreType.DMA] * DRING
            + [pltpu.SemaphoreType.DMA] * DRING
            + [pltpu.SemaphoreType.DMA]
        ),
    )
    def body(w_hbm, pos_hbm, x_hbm, xs_hbm, roww_hbm, *rest):
        w_v, pos_v, w16 = rest[:3]
        bufs = rest[3:3 + DRING]
        rsems = rest[3 + DRING:3 + 2 * DRING]
        ssems = rest[3 + 2 * DRING:3 + 3 * DRING]
        wsem = rest[3 + 3 * DRING]
        wid = lax.axis_index("s") * NC + lax.axis_index("c")
        pbase = wid * P_PER_W
        tb = lax.rem(wid, 16) * P_PER_W
        pltpu.sync_copy(w_hbm.at[pl.ds(pbase, P_PER_W)], w_v)
        pltpu.sync_copy(pos_hbm.at[pl.ds(wid * NCH_D, NCH_D)], pos_v)
        for c in range(NCH_D):
            w16[c, :] = w_v[pl.ds(c * DCH, DCH)]
        wds = []
        for c in range(NCH_D):
            wds.append(pltpu.async_copy(
                w16.at[c], roww_hbm.at[pos_v.at[c]], wsem))
        read_d = [None] * DRING
        scat_d = [None] * DRING
        for c in range(NCH_D):
            rb = c % DRING
            if scat_d[rb] is not None:
                scat_d[rb].wait()
                scat_d[rb] = None
            read_d[rb] = pltpu.async_copy(
                x_hbm.at[pl.ds(tb + c * DCH, DCH)], bufs[rb], rsems[rb])
            read_d[rb].wait()
            scat_d[rb] = pltpu.async_copy(
                bufs[rb], xs_hbm.at[pos_v.at[c]], ssems[rb])
        for rb in range(DRING):
            if scat_d[rb] is not None:
                scat_d[rb].wait()
        for d in wds:
            d.wait()

    return body(w_flat, pos2d, flat_x)


def _gmm_body(tile_e_ref, xs_ref, we_ref, be_ref, wv_ref, ys_ref):
    del tile_e_ref
    acc = jnp.dot(xs_ref[...], we_ref[0], preferred_element_type=jnp.float32)
    acc = acc + be_ref[0, 0][None, :]
    ys_ref[...] = acc * wv_ref[0, 0][:, None]


def _grouped_matmul(tile_e, xs, We, be, row_w):
    grid_spec = pltpu.PrefetchScalarGridSpec(
        num_scalar_prefetch=1,
        grid=(NT,),
        in_specs=[
            pl.BlockSpec((TILE_M, DIN), lambda i, te: (i, 0)),
            pl.BlockSpec((1, DIN, DOUT), lambda i, te: (te[i], 0, 0)),
            pl.BlockSpec((1, 1, DOUT), lambda i, te: (te[i], 0, 0)),
            pl.BlockSpec((1, 1, TILE_M), lambda i, te: (i, 0, 0)),
        ],
        out_specs=pl.BlockSpec((TILE_M, DOUT), lambda i, te: (i, 0)),
    )
    return pl.pallas_call(
        _gmm_body,
        grid_spec=grid_spec,
        out_shape=jax.ShapeDtypeStruct((PAD_TOT, DOUT), jnp.float32),
    )(tile_e, xs, We, be.reshape(E, 1, DOUT), row_w.reshape(NT, 1, TILE_M))


CCH = 16
T_PER_W = T // NW


def _sc_combine(ys, pos0, pos1):
    nch = T_PER_W // CCH

    @functools.partial(
        pl.kernel,
        out_type=jax.ShapeDtypeStruct((T, DOUT), jnp.float32),
        mesh=_sc_mesh(),
        scratch_types=[
            pltpu.VMEM((T_PER_W,), jnp.int32),
            pltpu.VMEM((T_PER_W,), jnp.int32),
            pltpu.VMEM((CCH, DOUT), jnp.float32),
            pltpu.VMEM((CCH, DOUT), jnp.float32),
            pltpu.VMEM((CCH, DOUT), jnp.float32),
            pltpu.VMEM((CCH, DOUT), jnp.float32),
            pltpu.SemaphoreType.DMA,
            pltpu.SemaphoreType.DMA,
            pltpu.SemaphoreType.DMA,
            pltpu.SemaphoreType.DMA,
        ],
    )
    def body(ys_hbm, p0_hbm, p1_hbm, out_hbm,
             p0_all, p1_all, a0, b0_, a1, b1_, g0, g1, s0, s1):
        wid = lax.axis_index("s") * NC + lax.axis_index("c")
        base = wid * T_PER_W
        nvec = DOUT // 16
        half = CCH // 2
        pltpu.sync_copy(p0_hbm.at[pl.ds(base, T_PER_W)], p0_all)
        pltpu.sync_copy(p1_hbm.at[pl.ds(base, T_PER_W)], p1_all)
        abufs, bbufs, gsems, wsems = (a0, a1), (b0_, b1_), (g0, g1), (s0, s1)
        gather_d = [[], []]
        write_d = [None, None]

        def fire_gathers(c):
            b = c % 2
            for h in range(2):
                gather_d[b].append(pltpu.async_copy(
                    ys_hbm.at[p0_all.at[pl.ds(c * CCH + h * half, half)]],
                    abufs[b].at[pl.ds(h * half, half)], gsems[b]))
                gather_d[b].append(pltpu.async_copy(
                    ys_hbm.at[p1_all.at[pl.ds(c * CCH + h * half, half)]],
                    bbufs[b].at[pl.ds(h * half, half)], gsems[b]))

        fire_gathers(0)
        for c in range(nch):
            b = c % 2
            if c + 1 < nch:
                nb = (c + 1) % 2
                if write_d[nb] is not None:
                    write_d[nb].wait()
                    write_d[nb] = None
                fire_gathers(c + 1)
            for d in gather_d[b]:
                d.wait()
            gather_d[b] = []
            av, bv = abufs[b], bbufs[b]

            def add_body(r, _):
                for v in range(nvec):
                    col = v * 16
                    av[r, pl.ds(col, 16)] = (
                        av[r, pl.ds(col, 16)] + bv[r, pl.ds(col, 16)])
                return 0

            lax.fori_loop(0, CCH, add_body, 0)
            write_d[b] = pltpu.async_copy(
                av, out_hbm.at[pl.ds(base + c * CCH, CCH)], wsems[b])
        for b in range(2):
            if write_d[b] is not None:
                write_d[b].wait()

    return body(ys, pos0, pos1)


def kernel(x, Wr, br, We, be):
    seq, batch, _ = x.shape
    flat_x = x.reshape(T, DIN)
    i0, i1, w0, w1, cnt, rank4 = _router(flat_x, Wr, br)
    cnt32 = jnp.concatenate(
        [cnt[:, :2].reshape(16, E), cnt[:, 2:].reshape(16, E)], axis=0)
    rank_flat = jnp.concatenate(
        [rank4[:, :2].reshape(16 * P_PER_W), rank4[:, 2:].reshape(16 * P_PER_W)])
    e_flat = jnp.concatenate([i0, i1])
    w_flat = jnp.concatenate([w0, w1])
    tot = cnt32.sum(axis=0)
    padded = ((tot + TILE_M - 1) // TILE_M) * TILE_M
    off = jnp.concatenate([jnp.zeros((1,), jnp.int32),
                           jnp.cumsum(padded)[:-1].astype(jnp.int32)])
    start32 = jnp.cumsum(cnt32, axis=0) - cnt32
    base_flat = (off[None, :] + start32).reshape(NW * E)
    widx = jnp.arange(K * T, dtype=jnp.int32) // P_PER_W
    posflat = base_flat[widx * E + e_flat] + rank_flat
    xs, row_w = _sc_dispatch(w_flat, posflat.reshape(NW * NCH_D, DCH), flat_x)
    ends = off + padded
    tile_starts = jnp.arange(NT, dtype=jnp.int32) * TILE_M
    tile_e = jnp.minimum(
        (tile_starts[:, None] >= ends[None, :]).sum(axis=1), E - 1
    ).astype(jnp.int32)
    ys = _grouped_matmul(tile_e, xs, We, be, row_w)
    out = _sc_combine(ys, posflat[:T], posflat[T:])
    return out.reshape(seq, batch, DOUT)


# speedup vs baseline: 1.0045x; 1.0045x over previous
import functools

import jax
import jax.numpy as jnp
from jax import lax
from jax.experimental import pallas as pl
from jax.experimental.pallas import tpu as pltpu
from jax.experimental.pallas import tpu_sc as plsc

E = 8
K = 2
T = 4096
DIN = 1024
DOUT = 1024
TILE_M = 256
PAD_TOT = T * K + E * TILE_M
NT = PAD_TOT // TILE_M

NC, NS = 2, 16
NW = NC * NS
P_PER_W = K * T // NW

ROUTER_TILE = 512


def _sc_mesh():
    return plsc.VectorSubcoreMesh(
        core_axis_name="c", subcore_axis_name="s", num_cores=NC, num_subcores=NS)


def _router_body(x_ref, wr_ref, br_ref,
                 i0_ref, i1_ref, w0_ref, w1_ref, cnt_ref, rank_ref):
    logits = jnp.dot(x_ref[...], wr_ref[...], preferred_element_type=jnp.float32)
    logits = logits + br_ref[...]
    ids = lax.broadcasted_iota(jnp.int32, logits.shape, 1)
    m0 = jnp.max(logits, axis=1, keepdims=True)
    i0 = jnp.min(jnp.where(logits == m0, ids, E), axis=1)
    masked = jnp.where(ids == i0[:, None], -jnp.inf, logits)
    m1 = jnp.max(masked, axis=1, keepdims=True)
    i1 = jnp.min(jnp.where(masked == m1, ids, E), axis=1)
    d = jnp.exp(m1[:, 0] - m0[:, 0])
    i0_ref[...] = i0
    i1_ref[...] = i1
    w0_ref[...] = 1.0 / (1.0 + d)
    w1_ref[...] = d / (1.0 + d)
    eids = lax.broadcasted_iota(jnp.int32, (P_PER_W, E), 1)
    tri = (lax.broadcasted_iota(jnp.int32, (P_PER_W, P_PER_W), 0)
           > lax.broadcasted_iota(jnp.int32, (P_PER_W, P_PER_W), 1)
           ).astype(jnp.float32)
    for h in range(2):
        sl = slice(h * P_PER_W, (h + 1) * P_PER_W)
        for k, ik in ((0, i0), (1, i1)):
            onehot = (ik[sl][:, None] == eids).astype(jnp.float32)
            cnt_ref[0, 2 * k + h] = jnp.sum(onehot, axis=0).astype(jnp.int32)
            before = jnp.dot(tri, onehot, preferred_element_type=jnp.float32)
            rank_ref[0, 2 * k + h] = jnp.sum(
                before * onehot, axis=1).astype(jnp.int32)


def _router(flat_x, Wr, br):
    nblk = T // ROUTER_TILE
    out_shapes = (
        jax.ShapeDtypeStruct((T,), jnp.int32),
        jax.ShapeDtypeStruct((T,), jnp.int32),
        jax.ShapeDtypeStruct((T,), jnp.float32),
        jax.ShapeDtypeStruct((T,), jnp.float32),
        jax.ShapeDtypeStruct((nblk, 4, E), jnp.int32),
        jax.ShapeDtypeStruct((nblk, 4, P_PER_W), jnp.int32),
    )
    vec_spec = pl.BlockSpec((ROUTER_TILE,), lambda i: (i,))
    return pl.pallas_call(
        _router_body,
        grid=(nblk,),
        in_specs=[
            pl.BlockSpec((ROUTER_TILE, DIN), lambda i: (i, 0)),
            pl.BlockSpec((DIN, E), lambda i: (0, 0)),
            pl.BlockSpec((1, E), lambda i: (0, 0)),
        ],
        out_specs=(vec_spec, vec_spec, vec_spec, vec_spec,
                   pl.BlockSpec((1, 4, E), lambda i: (i, 0, 0)),
                   pl.BlockSpec((1, 4, P_PER_W), lambda i: (i, 0, 0))),
        out_shape=out_shapes,
    )(flat_x, Wr, br.reshape(1, E))


DCH = 16
NCH_D = P_PER_W // DCH
DRING = 6


def _sc_dispatch(w_flat, pos2d, flat_x):
    @functools.partial(
        pl.kernel,
        out_type=(
            jax.ShapeDtypeStruct((PAD_TOT, DIN), jnp.float32),
            jax.ShapeDtypeStruct((PAD_TOT,), jnp.float32),
        ),
        mesh=_sc_mesh(),
        scratch_types=(
            [pltpu.VMEM((P_PER_W,), jnp.float32),
             pltpu.VMEM((NCH_D, DCH), jnp.int32),
             pltpu.VMEM((NCH_D, DCH), jnp.float32)]
            + [pltpu.VMEM((DCH, DIN), jnp.float32)] * DRING
            + [pltpu.SemaphoreType.DMA] * DRING
            + [pltpu.SemaphoreType.DMA] * DRING
            + [pltpu.SemaphoreType.DMA]
        ),
    )
    def body(w_hbm, pos_hbm, x_hbm, xs_hbm, roww_hbm, *rest):
        w_v, pos_v, w16 = rest[:3]
        bufs = rest[3:3 + DRING]
        rsems = rest[3 + DRING:3 + 2 * DRING]
        ssems = rest[3 + 2 * DRING:3 + 3 * DRING]
        wsem = rest[3 + 3 * DRING]
        wid = lax.axis_index("s") * NC + lax.axis_index("c")
        pbase = wid * P_PER_W
        tb = lax.rem(wid, 16) * P_PER_W
        pltpu.sync_copy(w_hbm.at[pl.ds(pbase, P_PER_W)], w_v)
        pltpu.sync_copy(pos_hbm.at[pl.ds(wid * NCH_D, NCH_D)], pos_v)
        for c in range(NCH_D):
            w16[c, :] = w_v[pl.ds(c * DCH, DCH)]
        wds = []
        for c in range(NCH_D):
            wds.append(pltpu.async_copy(
                w16.at[c], roww_hbm.at[pos_v.at[c]], wsem))
        read_d = [None] * DRING
        scat_d = [None] * DRING
        for c in range(NCH_D):
            rb = c % DRING
            if scat_d[rb] is not None:
                scat_d[rb].wait()
                scat_d[rb] = None
            read_d[rb] = pltpu.async_copy(
                x_hbm.at[pl.ds(tb + c * DCH, DCH)], bufs[rb], rsems[rb])
            read_d[rb].wait()
            scat_d[rb] = pltpu.async_copy(
                bufs[rb], xs_hbm.at[pos_v.at[c]], ssems[rb])
        for rb in range(DRING):
            if scat_d[rb] is not None:
                scat_d[rb].wait()
        for d in wds:
            d.wait()

    return body(w_flat, pos2d, flat_x)


def _gmm_body(tile_e_ref, xs_ref, we_ref, be_ref, wv_ref, ys_ref):
    del tile_e_ref
    acc = jnp.dot(xs_ref[...].astype(jnp.bfloat16),
                  we_ref[0].astype(jnp.bfloat16),
                  preferred_element_type=jnp.float32)
    acc = acc + be_ref[0, 0][None, :]
    ys_ref[...] = acc * wv_ref[0, 0][:, None]


def _grouped_matmul(tile_e, xs, We, be, row_w):
    grid_spec = pltpu.PrefetchScalarGridSpec(
        num_scalar_prefetch=1,
        grid=(NT,),
        in_specs=[
            pl.BlockSpec((TILE_M, DIN), lambda i, te: (i, 0)),
            pl.BlockSpec((1, DIN, DOUT), lambda i, te: (te[i], 0, 0)),
            pl.BlockSpec((1, 1, DOUT), lambda i, te: (te[i], 0, 0)),
            pl.BlockSpec((1, 1, TILE_M), lambda i, te: (i, 0, 0)),
        ],
        out_specs=pl.BlockSpec((TILE_M, DOUT), lambda i, te: (i, 0)),
    )
    return pl.pallas_call(
        _gmm_body,
        grid_spec=grid_spec,
        out_shape=jax.ShapeDtypeStruct((PAD_TOT, DOUT), jnp.float32),
    )(tile_e, xs, We, be.reshape(E, 1, DOUT), row_w.reshape(NT, 1, TILE_M))


CCH = 16
T_PER_W = T // NW


def _sc_combine(ys, pos0, pos1):
    nch = T_PER_W // CCH

    @functools.partial(
        pl.kernel,
        out_type=jax.ShapeDtypeStruct((T, DOUT), jnp.float32),
        mesh=_sc_mesh(),
        scratch_types=[
            pltpu.VMEM((T_PER_W,), jnp.int32),
            pltpu.VMEM((T_PER_W,), jnp.int32),
            pltpu.VMEM((CCH, DOUT), jnp.float32),
            pltpu.VMEM((CCH, DOUT), jnp.float32),
            pltpu.VMEM((CCH, DOUT), jnp.float32),
            pltpu.VMEM((CCH, DOUT), jnp.float32),
            pltpu.SemaphoreType.DMA,
            pltpu.SemaphoreType.DMA,
            pltpu.SemaphoreType.DMA,
            pltpu.SemaphoreType.DMA,
        ],
    )
    def body(ys_hbm, p0_hbm, p1_hbm, out_hbm,
             p0_all, p1_all, a0, b0_, a1, b1_, g0, g1, s0, s1):
        wid = lax.axis_index("s") * NC + lax.axis_index("c")
        base = wid * T_PER_W
        nvec = DOUT // 16
        half = CCH // 2
        pltpu.sync_copy(p0_hbm.at[pl.ds(base, T_PER_W)], p0_all)
        pltpu.sync_copy(p1_hbm.at[pl.ds(base, T_PER_W)], p1_all)
        abufs, bbufs, gsems, wsems = (a0, a1), (b0_, b1_), (g0, g1), (s0, s1)
        gather_d = [[], []]
        write_d = [None, None]

        def fire_gathers(c):
            b = c % 2
            for h in range(2):
                gather_d[b].append(pltpu.async_copy(
                    ys_hbm.at[p0_all.at[pl.ds(c * CCH + h * half, half)]],
                    abufs[b].at[pl.ds(h * half, half)], gsems[b]))
                gather_d[b].append(pltpu.async_copy(
                    ys_hbm.at[p1_all.at[pl.ds(c * CCH + h * half, half)]],
                    bbufs[b].at[pl.ds(h * half, half)], gsems[b]))

        fire_gathers(0)
        for c in range(nch):
            b = c % 2
            if c + 1 < nch:
                nb = (c + 1) % 2
                if write_d[nb] is not None:
                    write_d[nb].wait()
                    write_d[nb] = None
                fire_gathers(c + 1)
            for d in gather_d[b]:
                d.wait()
            gather_d[b] = []
            av, bv = abufs[b], bbufs[b]

            def add_body(r, _):
                for v in range(nvec):
                    col = v * 16
                    av[r, pl.ds(col, 16)] = (
                        av[r, pl.ds(col, 16)] + bv[r, pl.ds(col, 16)])
                return 0

            lax.fori_loop(0, CCH, add_body, 0)
            write_d[b] = pltpu.async_copy(
                av, out_hbm.at[pl.ds(base + c * CCH, CCH)], wsems[b])
        for b in range(2):
            if write_d[b] is not None:
                write_d[b].wait()

    return body(ys, pos0, pos1)


def kernel(x, Wr, br, We, be):
    seq, batch, _ = x.shape
    flat_x = x.reshape(T, DIN)
    i0, i1, w0, w1, cnt, rank4 = _router(flat_x, Wr, br)
    cnt32 = jnp.concatenate(
        [cnt[:, :2].reshape(16, E), cnt[:, 2:].reshape(16, E)], axis=0)
    rank_flat = jnp.concatenate(
        [rank4[:, :2].reshape(16 * P_PER_W), rank4[:, 2:].reshape(16 * P_PER_W)])
    e_flat = jnp.concatenate([i0, i1])
    w_flat = jnp.concatenate([w0, w1])
    tot = cnt32.sum(axis=0)
    padded = ((tot + TILE_M - 1) // TILE_M) * TILE_M
    off = jnp.concatenate([jnp.zeros((1,), jnp.int32),
                           jnp.cumsum(padded)[:-1].astype(jnp.int32)])
    start32 = jnp.cumsum(cnt32, axis=0) - cnt32
    base_flat = (off[None, :] + start32).reshape(NW * E)
    widx = jnp.arange(K * T, dtype=jnp.int32) // P_PER_W
    posflat = base_flat[widx * E + e_flat] + rank_flat
    xs, row_w = _sc_dispatch(w_flat, posflat.reshape(NW * NCH_D, DCH), flat_x)
    ends = off + padded
    tile_starts = jnp.arange(NT, dtype=jnp.int32) * TILE_M
    tile_e = jnp.minimum(
        (tile_starts[:, None] >= ends[None, :]).sum(axis=1), E - 1
    ).astype(jnp.int32)
    ys = _grouped_matmul(tile_e, xs, We, be, row_w)
    out = _sc_combine(ys, posflat[:T], posflat[T:])
    return out.reshape(seq, batch, DOUT)
